# baseline (speedup 1.0000x reference)
import functools

import numpy as np
import jax
import jax.numpy as jnp
from jax import lax
from jax.experimental import pallas as pl
from jax.experimental.pallas import tpu as pltpu
from jax.experimental.pallas import tpu_sc as plsc

_LANES = 16


def _positional_encoding_np(d_model, length):
    pos = np.arange(length, dtype=np.float32)[:, None]
    div = np.exp(
        np.arange(0, d_model, 2, dtype=np.float32) * (-np.log(10000.0) / d_model)
    )
    pe = np.zeros((length, d_model), dtype=np.float32)
    pe[:, 0::2] = np.sin(pos * div)
    pe[:, 1::2] = np.cos(pos * div)
    return pe


def kernel(x, table):
    B, L = x.shape
    V, D = table.shape

    NC, NS = 2, 16
    NW = NC * NS

    TB = 256
    NTC = TB // 128
    NBB = B // TB
    NU = (L * NBB) // NW
    assert NU * NW == L * NBB
    TRW = (D // 8) * NTC * 8 * 128
    PIECE = NTC * 8 * 128

    xT = x.astype(jnp.int32).T.reshape(L * B)
    pe = jnp.asarray(_positional_encoding_np(D, L))

    mesh = plsc.VectorSubcoreMesh(core_axis_name="c", subcore_axis_name="s")

    @functools.partial(
        pl.kernel,
        mesh=mesh,
        compiler_params=pltpu.CompilerParams(
            use_tc_tiling_on_sc=False, needs_layout_passes=False
        ),
        out_type=jax.ShapeDtypeStruct((L * D * B,), jnp.float32),
        scratch_types=[
            pltpu.VMEM((L, D), jnp.float32),
            pltpu.VMEM((2, TB), jnp.int32),
            pltpu.VMEM((2 * TB, D), jnp.float32),
            pltpu.VMEM((2 * TRW,), jnp.float32),
            pltpu.SemaphoreType.DMA((2,)),
            pltpu.SemaphoreType.DMA((2,)),
        ],
    )
    def run(xf_hbm, table_hbm, pe_hbm, out_hbm, pe_v, idx_v, rows_v, tr_v, gsem, osem):
        wid = lax.axis_index("s") * NC + lax.axis_index("c")
        pltpu.sync_copy(pe_hbm, pe_v)
        iota = lax.iota(jnp.int32, _LANES)
        consts = []
        for q in range(D // _LANES):
            dvec = iota + q * _LANES
            consts.append((dvec // 8) * PIECE + (dvec % 8) * 128)

        def fetch(t):
            u = wid * NU + t
            l = u // NBB
            bb = u % NBB
            s = t % 2
            pltpu.sync_copy(xf_hbm.at[pl.ds(l * B + bb * TB, TB)], idx_v.at[s])
            for k in range(TB // 128):
                pltpu.async_copy(
                    table_hbm.at[idx_v.at[s, pl.ds(k * 128, 128)]],
                    rows_v.at[pl.ds(s * TB + k * 128, 128), pl.ds(0, D)],
                    gsem.at[s],
                )

        fetch(0)

        def unit_body(t, carry):
            u = wid * NU + t
            l = u // NBB
            bb = u % NBB
            s = t % 2

            @pl.when(t + 1 < NU)
            def _():
                fetch(t + 1)

            for k in range(TB // 128):
                pltpu.make_async_copy(
                    table_hbm.at[idx_v.at[s, pl.ds(0, 128)]],
                    rows_v.at[pl.ds(0, 128), pl.ds(0, D)],
                    gsem.at[s],
                ).wait()

            @pl.when(t >= 2)
            def _():
                for _p in range(D // 8):
                    pltpu.make_async_copy(
                        tr_v.at[pl.ds(0, PIECE)],
                        out_hbm.at[pl.ds(0, PIECE)],
                        osem.at[s],
                    ).wait()

            for q in range(D // _LANES):
                pe_q = pe_v[l, pl.ds(q * _LANES, _LANES)]
                cst_q = consts[q] + s * TRW

                @plsc.parallel_loop(0, TB, unroll=4)
                def _(j, _q=q, _pe=pe_q, _cst=cst_q):
                    base = (j // 128) * 1024 + (j % 128)
                    bvec = jnp.full((_LANES,), base, jnp.int32)
                    v = rows_v[s * TB + j, pl.ds(_q * _LANES, _LANES)]
                    plsc.store_scatter(tr_v, [_cst + bvec], v + _pe)

            obase = ((l * (D // 8)) * (B // 128) + bb * NTC) * 1024
            for p in range(D // 8):
                pltpu.async_copy(
                    tr_v.at[pl.ds(s * TRW + p * PIECE, PIECE)],
                    out_hbm.at[pl.ds(obase + p * (B // 128) * 1024, PIECE)],
                    osem.at[s],
                )
            return carry

        lax.fori_loop(0, NU, unit_body, 0)

        for s in range(2):
            if NU >= 2 - s:
                for _p in range(D // 8):
                    pltpu.make_async_copy(
                        tr_v.at[pl.ds(0, PIECE)],
                        out_hbm.at[pl.ds(0, PIECE)],
                        osem.at[s],
                    ).wait()

    out_flat = run(xT, table, pe)
    out5 = out_flat.reshape(L, D // 8, B // 128, 8, 128)
    return out5.transpose(2, 4, 0, 1, 3).reshape(B, L, D)


# speedup vs baseline: 1.4626x; 1.4626x over previous
import functools

import numpy as np
import jax
import jax.numpy as jnp
from jax import lax
from jax.experimental import pallas as pl
from jax.experimental.pallas import tpu as pltpu
from jax.experimental.pallas import tpu_sc as plsc

_LANES = 16


def _positional_encoding_np(d_model, length):
    pos = np.arange(length, dtype=np.float32)[:, None]
    div = np.exp(
        np.arange(0, d_model, 2, dtype=np.float32) * (-np.log(10000.0) / d_model)
    )
    pe = np.zeros((length, d_model), dtype=np.float32)
    pe[:, 0::2] = np.sin(pos * div)
    pe[:, 1::2] = np.cos(pos * div)
    return pe


def kernel(x, table):
    B, L = x.shape
    V, D = table.shape

    NC, NS = 2, 16
    NW = NC * NS

    TB = 256
    NTC = TB // 128
    NBB = B // TB
    NU = (L * NBB) // NW
    assert NU * NW == L * NBB
    TRW = (D // 8) * NTC * 8 * 128
    PIECE = NTC * 8 * 128

    xT = x.astype(jnp.int32).T.reshape(L * B)
    pe = jnp.asarray(_positional_encoding_np(D, L))

    mesh = plsc.VectorSubcoreMesh(core_axis_name="c", subcore_axis_name="s")

    @functools.partial(
        pl.kernel,
        mesh=mesh,
        compiler_params=pltpu.CompilerParams(
            use_tc_tiling_on_sc=False, needs_layout_passes=False
        ),
        out_type=jax.ShapeDtypeStruct((L * D * B,), jnp.float32),
        scratch_types=[
            pltpu.VMEM((L, D), jnp.float32),
            pltpu.VMEM((2, TB), jnp.int32),
            pltpu.VMEM((2 * TB, D), jnp.float32),
            pltpu.VMEM((2 * TRW,), jnp.float32),
            pltpu.VMEM((128 * 17,), jnp.float32),
            pltpu.SemaphoreType.DMA((2,)),
            pltpu.SemaphoreType.DMA((2,)),
        ],
    )
    def run(xf_hbm, table_hbm, pe_hbm, out_hbm, pe_v, idx_v, rows_v, tr_v, stage_v, gsem, osem):
        wid = lax.axis_index("s") * NC + lax.axis_index("c")
        pltpu.sync_copy(pe_hbm, pe_v)
        iota = lax.iota(jnp.int32, _LANES)
        iota17 = iota * 17

        def fetch(t):
            u = wid * NU + t
            l = u // NBB
            bb = u % NBB
            s = t % 2
            pltpu.sync_copy(xf_hbm.at[pl.ds(l * B + bb * TB, TB)], idx_v.at[s])
            for k in range(TB // 128):
                pltpu.async_copy(
                    table_hbm.at[idx_v.at[s, pl.ds(k * 128, 128)]],
                    rows_v.at[pl.ds(s * TB + k * 128, 128), pl.ds(0, D)],
                    gsem.at[s],
                )

        fetch(0)

        def unit_body(t, carry):
            u = wid * NU + t
            l = u // NBB
            bb = u % NBB
            s = t % 2

            @pl.when(t + 1 < NU)
            def _():
                fetch(t + 1)

            for k in range(TB // 128):
                pltpu.make_async_copy(
                    table_hbm.at[idx_v.at[s, pl.ds(0, 128)]],
                    rows_v.at[pl.ds(0, 128), pl.ds(0, D)],
                    gsem.at[s],
                ).wait()

            @pl.when(t >= 2)
            def _():
                for _p in range(D // 8):
                    pltpu.make_async_copy(
                        tr_v.at[pl.ds(0, PIECE)],
                        out_hbm.at[pl.ds(0, PIECE)],
                        osem.at[s],
                    ).wait()

            for half in range(TB // 128):
                for q in range(D // _LANES):
                    pe_q = pe_v[l, pl.ds(q * _LANES, _LANES)]
                    rbase = s * TB + half * 128
                    obase0 = s * TRW + 2 * q * PIECE + half * 1024

                    @plsc.parallel_loop(0, 128, unroll=4)
                    def _(j, _pe=pe_q, _rb=rbase, _q=q):
                        v = rows_v[_rb + j, pl.ds(_q * _LANES, _LANES)]
                        stage_v[pl.ds(j * 17, _LANES)] = v + _pe

                    @plsc.parallel_loop(0, 128, unroll=4)
                    def _(i, _ob=obase0):
                        c = i // 8
                        mg = i % 8
                        col = plsc.load_gather(
                            stage_v, [iota17 + (mg * (_LANES * 17) + c)]
                        )
                        off = _ob + (c // 8) * PIECE + (c % 8) * 128 + mg * _LANES
                        tr_v[pl.ds(off, _LANES)] = col

            obase = ((l * (D // 8)) * (B // 128) + bb * NTC) * 1024
            for p in range(D // 8):
                pltpu.async_copy(
                    tr_v.at[pl.ds(s * TRW + p * PIECE, PIECE)],
                    out_hbm.at[pl.ds(obase + p * (B // 128) * 1024, PIECE)],
                    osem.at[s],
                )
            return carry

        lax.fori_loop(0, NU, unit_body, 0)

        for s in range(2):
            if NU >= 2 - s:
                for _p in range(D // 8):
                    pltpu.make_async_copy(
                        tr_v.at[pl.ds(0, PIECE)],
                        out_hbm.at[pl.ds(0, PIECE)],
                        osem.at[s],
                    ).wait()

    out_flat = run(xT, table, pe)
    out5 = out_flat.reshape(L, D // 8, B // 128, 8, 128)
    return out5.transpose(2, 4, 0, 1, 3).reshape(B, L, D)
